# baseline (speedup 1.0000x reference)
import functools

import jax
import jax.numpy as jnp
from jax import lax
from jax.experimental import pallas as pl
from jax.experimental.pallas import tpu as pltpu
from jax.experimental.pallas import tpu_sc as plsc

VOCAB = 1000000
DIM = 64
B = 4096
L = 200

NUM_CORES = 2
NUM_SUBCORES = 16
NW = NUM_CORES * NUM_SUBCORES
B_PER_W = B // NW
IDX_PER_W = B_PER_W * L
BPD = 2
DROWS = BPD * L
NDESC = B_PER_W // BPD
LANES = 16
NCH = DIM // LANES

_mesh = plsc.VectorSubcoreMesh(
    core_axis_name="c", subcore_axis_name="s",
    num_cores=NUM_CORES, num_subcores=NUM_SUBCORES)


@functools.partial(
    pl.kernel,
    out_type=jax.ShapeDtypeStruct((B, DIM), jnp.float32),
    mesh=_mesh,
    scratch_types=[
        pltpu.VMEM((IDX_PER_W,), jnp.int32),
        pltpu.VMEM((DROWS, DIM), jnp.float32),
        pltpu.VMEM((DROWS, DIM), jnp.float32),
        pltpu.VMEM((DROWS, DIM), jnp.float32),
        pltpu.VMEM((B_PER_W, DIM), jnp.float32),
        pltpu.SemaphoreType.DMA,
        pltpu.SemaphoreType.DMA,
        pltpu.SemaphoreType.DMA,
    ],
    compiler_params=pltpu.CompilerParams(use_tc_tiling_on_sc=False),
)
def _embed_mean(idx_hbm, table_hbm, out_hbm, idx_v, rows0, rows1, rows2,
                out_v, sem0, sem1, sem2):
    wid = lax.axis_index("s") * NUM_CORES + lax.axis_index("c")

    pltpu.sync_copy(idx_hbm.at[pl.ds(wid * IDX_PER_W, IDX_PER_W)], idx_v)

    def issue(buf, sem, g):
        for j in range(DROWS // LANES):
            vec = idx_v[pl.ds(g * DROWS + j * LANES, LANES)]
            pltpu.async_copy(table_hbm.at[vec],
                             buf.at[pl.ds(j * LANES, LANES)], sem)

    def wait(buf, sem):
        pltpu.make_async_copy(table_hbm.at[pl.ds(0, DROWS)], buf, sem).wait()

    ring = ((rows0, sem0, 0), (rows1, sem1, 1), (rows2, sem2, 2))
    NBUF = len(ring)
    for buf, sem, off in ring:
        issue(buf, sem, off)

    scale = jnp.float32(1.0 / L)
    UNROLL = 8

    def accumulate(buf, g):
        for j in range(BPD):
            b = g * BPD + j

            def acc_body(i, carry, j=j):
                acc = list(carry)
                base = j * L + i * UNROLL
                for r in range(UNROLL):
                    for c in range(NCH):
                        k = c * 2 + (r % 2)
                        acc[k] = acc[k] + buf[base + r,
                                              pl.ds(c * LANES, LANES)]
                return tuple(acc)

            zero = jnp.zeros((LANES,), jnp.float32)
            acc = lax.fori_loop(0, L // UNROLL, acc_body, (zero,) * (2 * NCH))
            for c in range(NCH):
                out_v[b, pl.ds(c * LANES, LANES)] = (
                    acc[c * 2] + acc[c * 2 + 1]) * scale

    def outer_body(i, carry):
        g0 = NBUF * i
        for buf, sem, off in ring:
            g = g0 + off
            wait(buf, sem)
            accumulate(buf, g)

            @pl.when(g + NBUF < NDESC)
            def _():
                issue(buf, sem, g + NBUF)
        return carry

    lax.fori_loop(0, NDESC // NBUF, outer_body, 0)

    REM = NDESC % NBUF
    for r in range(REM):
        buf, sem, _ = ring[r]
        wait(buf, sem)
        accumulate(buf, NDESC - REM + r)

    pltpu.sync_copy(out_v, out_hbm.at[pl.ds(wid * B_PER_W, B_PER_W)])


def kernel(indices, table):
    return _embed_mean(indices.reshape(-1), table)
